# baseline (speedup 1.0000x reference)
import functools

import jax
import jax.numpy as jnp
from jax import lax
from jax.experimental import pallas as pl
from jax.experimental.pallas import tpu as pltpu

TOPK_E = 64
TOPK_D = 768
TOPK_N = 32768
BLOCK_ROWS = 8192
NEG_INF = float("-inf")


def _gate_block(x_ref, w_ref, gates_ref, idx_ref, loss_ref, me_acc, ce_acc):
    i = pl.program_id(0)
    nsteps = pl.num_programs(0)

    x = x_ref[...]
    w = w_ref[...]
    logits = lax.dot_general(
        w, x, (((1,), (1,)), ((), ())),
        preferred_element_type=jnp.float32,
        precision=lax.Precision.DEFAULT,
    )

    iota = lax.broadcasted_iota(jnp.int32, logits.shape, 0)

    m1 = jnp.max(logits, axis=0, keepdims=True)
    idx1 = jnp.min(jnp.where(logits == m1, iota, TOPK_E), axis=0,
                   keepdims=True)
    masked = jnp.where(iota == idx1, NEG_INF, logits)
    m2 = jnp.max(masked, axis=0, keepdims=True)
    idx2 = jnp.min(jnp.where(masked == m2, iota, TOPK_E), axis=0,
                   keepdims=True)

    p = jnp.exp(logits - m1)
    z = jnp.sum(p, axis=0, keepdims=True)
    zinv = 1.0 / z
    g1 = zinv
    g2 = jnp.exp(m2 - m1) * zinv

    gates_ref[...] = jnp.concatenate([g1, g2], axis=0)
    idx_ref[...] = jnp.concatenate([idx1, idx2], axis=0)

    me_part = p * zinv
    ce_part = (iota == idx1).astype(jnp.float32)

    @pl.when(i == 0)
    def _init():
        me_acc[...] = me_part
        ce_acc[...] = ce_part

    @pl.when(i > 0)
    def _accum():
        me_acc[...] += me_part
        ce_acc[...] += ce_part

    @pl.when(i == nsteps - 1)
    def _finish():
        scale = TOPK_E / (float(TOPK_N) * float(TOPK_N))
        me = jnp.sum(me_acc[...], axis=1, keepdims=True)
        ce = jnp.sum(ce_acc[...], axis=1, keepdims=True)
        loss_ref[...] = jnp.sum(me * ce, keepdims=True)[:1, :1] * scale


@jax.jit
def kernel(x, W):
    n, d = x.shape
    e = W.shape[0]
    grid = n // BLOCK_ROWS
    gates, idx, loss = pl.pallas_call(
        _gate_block,
        grid=(grid,),
        in_specs=[
            pl.BlockSpec((BLOCK_ROWS, d), lambda i: (i, 0)),
            pl.BlockSpec((e, d), lambda i: (0, 0)),
        ],
        out_specs=[
            pl.BlockSpec((2, BLOCK_ROWS), lambda i: (0, i)),
            pl.BlockSpec((2, BLOCK_ROWS), lambda i: (0, i)),
            pl.BlockSpec((1, 1), lambda i: (0, 0)),
        ],
        out_shape=[
            jax.ShapeDtypeStruct((2, n), jnp.float32),
            jax.ShapeDtypeStruct((2, n), jnp.int32),
            jax.ShapeDtypeStruct((1, 1), jnp.float32),
        ],
        scratch_shapes=[
            pltpu.VMEM((e, BLOCK_ROWS), jnp.float32),
            pltpu.VMEM((e, BLOCK_ROWS), jnp.float32),
        ],
    )(x, W)
    return gates.T, loss[0, 0], idx.T


# speedup vs baseline: 1.0695x; 1.0695x over previous
import functools

import jax
import jax.numpy as jnp
from jax import lax
from jax.experimental import pallas as pl
from jax.experimental.pallas import tpu as pltpu

TOPK_E = 64
TOPK_D = 768
TOPK_N = 32768
BLOCK_ROWS = 4096
NEG_INF = float("-inf")


def _gate_block(x_ref, w_ref, gates_ref, idx_ref, loss_ref, me_acc, ce_acc):
    i = pl.program_id(0)
    nsteps = pl.num_programs(0)

    x = x_ref[...]
    w = w_ref[...]
    logits = lax.dot_general(
        w, x, (((1,), (1,)), ((), ())),
        preferred_element_type=jnp.float32,
        precision=lax.Precision.DEFAULT,
    )

    iota = lax.broadcasted_iota(jnp.int32, logits.shape, 0)

    m1 = jnp.max(logits, axis=0, keepdims=True)
    idx1 = jnp.min(jnp.where(logits == m1, iota, TOPK_E), axis=0,
                   keepdims=True)
    masked = jnp.where(iota == idx1, NEG_INF, logits)
    m2 = jnp.max(masked, axis=0, keepdims=True)
    idx2 = jnp.min(jnp.where(masked == m2, iota, TOPK_E), axis=0,
                   keepdims=True)

    p = jnp.exp(logits - m1)
    z = jnp.sum(p, axis=0, keepdims=True)
    zinv = 1.0 / z
    g1 = zinv
    g2 = jnp.exp(m2 - m1) * zinv

    gates_ref[...] = jnp.concatenate([g1, g2], axis=0)
    idx_ref[...] = jnp.concatenate([idx1, idx2], axis=0)

    me_part = p * zinv
    ce_part = (iota == idx1).astype(jnp.float32)

    @pl.when(i == 0)
    def _init():
        me_acc[...] = me_part
        ce_acc[...] = ce_part

    @pl.when(i > 0)
    def _accum():
        me_acc[...] += me_part
        ce_acc[...] += ce_part

    @pl.when(i == nsteps - 1)
    def _finish():
        scale = TOPK_E / (float(TOPK_N) * float(TOPK_N))
        me = jnp.sum(me_acc[...], axis=1, keepdims=True)
        ce = jnp.sum(ce_acc[...], axis=1, keepdims=True)
        loss_ref[...] = jnp.sum(me * ce, keepdims=True)[:1, :1] * scale


@jax.jit
def kernel(x, W):
    n, d = x.shape
    e = W.shape[0]
    grid = n // BLOCK_ROWS
    gates, idx, loss = pl.pallas_call(
        _gate_block,
        grid=(grid,),
        in_specs=[
            pl.BlockSpec((BLOCK_ROWS, d), lambda i: (i, 0)),
            pl.BlockSpec((e, d), lambda i: (0, 0)),
        ],
        out_specs=[
            pl.BlockSpec((2, BLOCK_ROWS), lambda i: (0, i)),
            pl.BlockSpec((2, BLOCK_ROWS), lambda i: (0, i)),
            pl.BlockSpec((1, 1), lambda i: (0, 0)),
        ],
        out_shape=[
            jax.ShapeDtypeStruct((2, n), jnp.float32),
            jax.ShapeDtypeStruct((2, n), jnp.int32),
            jax.ShapeDtypeStruct((1, 1), jnp.float32),
        ],
        scratch_shapes=[
            pltpu.VMEM((e, BLOCK_ROWS), jnp.float32),
            pltpu.VMEM((e, BLOCK_ROWS), jnp.float32),
        ],
    )(x, W)
    return gates.T, loss[0, 0], idx.T
